# Initial kernel scaffold; baseline (speedup 1.0000x reference)
import jax
import jax.numpy as jnp
from jax.experimental import pallas as pl


def kernel(coord, embeddings_table, special_token):
    raise NotImplementedError("write your pallas kernel here")


# speedup vs baseline: 6.1679x; 6.1679x over previous
import functools

import jax
import jax.numpy as jnp
from jax import lax
from jax.experimental import pallas as pl
from jax.experimental.pallas import tpu as pltpu
from jax.experimental.pallas import tpu_sc as plsc

_N_POS_EMB = 100000

_NC = 2
_NS = 16
_NW = _NC * _NS
_LANES = 16

_CHUNK = 2048
_GSUB = 128


def _make_gather(n, d):
    per_w = n // _NW
    n_chunks = per_w // _CHUNK
    mesh = plsc.VectorSubcoreMesh(core_axis_name="c", subcore_axis_name="s")

    @functools.partial(
        pl.kernel,
        mesh=mesh,
        compiler_params=pltpu.CompilerParams(use_tc_tiling_on_sc=False),
        out_type=jax.ShapeDtypeStruct((n, d), jnp.float32),
        scratch_types=[
            pltpu.VMEM((_CHUNK,), jnp.float32),
            pltpu.VMEM((_CHUNK,), jnp.int32),
            pltpu.VMEM((_CHUNK, d), jnp.float32),
            pltpu.SemaphoreType.DMA,
        ],
    )
    def gather_kernel(coord_hbm, table_hbm, out_hbm, coord_v, idx_v, rows_v, sem):
        wid = lax.axis_index("s") * _NC + lax.axis_index("c")

        def chunk_body(g, carry):
            base = wid * per_w + g * _CHUNK
            pltpu.sync_copy(coord_hbm.at[pl.ds(base, _CHUNK)], coord_v)

            def idx_body(i, carry2):
                c = coord_v[pl.ds(i * _LANES, _LANES)]
                pos = c * jnp.float32(_N_POS_EMB)
                pos = jnp.maximum(pos, jnp.float32(0.0))
                pos = jnp.minimum(pos, jnp.float32(_N_POS_EMB))
                idx_v[pl.ds(i * _LANES, _LANES)] = pos.astype(jnp.int32)
                return carry2

            lax.fori_loop(0, _CHUNK // _LANES, idx_body, 0)

            copies = [
                pltpu.async_copy(
                    table_hbm.at[idx_v.at[pl.ds(j * _GSUB, _GSUB)]],
                    rows_v.at[pl.ds(j * _GSUB, _GSUB)],
                    sem,
                )
                for j in range(_CHUNK // _GSUB)
            ]
            for cp in copies:
                cp.wait()

            pltpu.sync_copy(rows_v, out_hbm.at[pl.ds(base, _CHUNK)])
            return carry

        lax.fori_loop(0, n_chunks, chunk_body, 0)

    return gather_kernel


def kernel(coord, embeddings_table, special_token):
    b, h = coord.shape
    d = embeddings_table.shape[1]
    n = b * h
    flat = coord.reshape(n)
    out = _make_gather(n, d)(flat, embeddings_table)
    return out.reshape(b, h, d)


# speedup vs baseline: 6.5346x; 1.0595x over previous
import functools

import jax
import jax.numpy as jnp
from jax import lax
from jax.experimental import pallas as pl
from jax.experimental.pallas import tpu as pltpu
from jax.experimental.pallas import tpu_sc as plsc

_N_POS_EMB = 100000

_NC = 2
_NS = 16
_NW = _NC * _NS
_LANES = 16

_CHUNK = 2048
_GSUB = 128


def _make_gather(n, d):
    per_w = n // _NW
    n_chunks = per_w // _CHUNK
    assert n_chunks % 2 == 0
    mesh = plsc.VectorSubcoreMesh(core_axis_name="c", subcore_axis_name="s")

    @functools.partial(
        pl.kernel,
        mesh=mesh,
        compiler_params=pltpu.CompilerParams(use_tc_tiling_on_sc=False),
        out_type=jax.ShapeDtypeStruct((n, d), jnp.float32),
        scratch_types=[
            [pltpu.VMEM((_CHUNK,), jnp.float32) for _ in range(2)],
            [pltpu.VMEM((_CHUNK,), jnp.int32) for _ in range(2)],
            [pltpu.VMEM((_CHUNK, d), jnp.float32) for _ in range(2)],
            [pltpu.SemaphoreType.DMA for _ in range(2)],
            [pltpu.SemaphoreType.DMA for _ in range(2)],
            [pltpu.SemaphoreType.DMA for _ in range(2)],
        ],
    )
    def gather_kernel(coord_hbm, table_hbm, out_hbm, coord_v, idx_v, rows_v,
                      csem, gsem, osem):
        wid = lax.axis_index("s") * _NC + lax.axis_index("c")
        w_base = wid * per_w

        def coord_load(g, b):
            pltpu.async_copy(
                coord_hbm.at[pl.ds(w_base + g * _CHUNK, _CHUNK)], coord_v[b],
                csem[b])

        def coord_wait(b):
            pltpu.make_async_copy(
                coord_hbm.at[pl.ds(w_base, _CHUNK)], coord_v[b],
                csem[b]).wait()

        def quantize(b):
            @plsc.parallel_loop(0, _CHUNK, step=_LANES, unroll=8)
            def _(i):
                c = coord_v[b][pl.ds(i, _LANES)]
                pos = c * jnp.float32(_N_POS_EMB)
                pos = jnp.maximum(pos, jnp.float32(0.0))
                pos = jnp.minimum(pos, jnp.float32(_N_POS_EMB))
                idx_v[b][pl.ds(i, _LANES)] = pos.astype(jnp.int32)

        def fire_gathers(b):
            for j in range(_CHUNK // _GSUB):
                pltpu.async_copy(
                    table_hbm.at[idx_v[b].at[pl.ds(j * _GSUB, _GSUB)]],
                    rows_v[b].at[pl.ds(j * _GSUB, _GSUB)],
                    gsem[b])

        def gather_wait(b):
            pltpu.make_async_copy(
                out_hbm.at[pl.ds(w_base, _CHUNK)], rows_v[b], gsem[b]).wait()

        def fire_scatter(g, b):
            pltpu.async_copy(
                rows_v[b], out_hbm.at[pl.ds(w_base + g * _CHUNK, _CHUNK)],
                osem[b])

        def scatter_wait(b):
            pltpu.make_async_copy(
                rows_v[b], out_hbm.at[pl.ds(w_base, _CHUNK)], osem[b]).wait()

        coord_load(0, 0)
        coord_load(1, 1)

        def pair_body(p, carry):
            for b in range(2):
                g = 2 * p + b

                coord_wait(b)

                @pl.when(g >= 2)
                def _():
                    scatter_wait(b)

                quantize(b)

                @pl.when(g + 2 < n_chunks)
                def _():
                    coord_load(g + 2, b)

                fire_gathers(b)

                @pl.when(g >= 1)
                def _():
                    gather_wait(1 - b)
                    fire_scatter(g - 1, 1 - b)
            return carry

        lax.fori_loop(0, n_chunks // 2, pair_body, 0)

        gather_wait(1)
        fire_scatter(n_chunks - 1, 1)
        scatter_wait(0)
        scatter_wait(1)

    return gather_kernel


def kernel(coord, embeddings_table, special_token):
    b, h = coord.shape
    d = embeddings_table.shape[1]
    n = b * h
    flat = coord.reshape(n)
    out = _make_gather(n, d)(flat, embeddings_table)
    return out.reshape(b, h, d)


# speedup vs baseline: 6.7075x; 1.0265x over previous
import functools

import jax
import jax.numpy as jnp
from jax import lax
from jax.experimental import pallas as pl
from jax.experimental.pallas import tpu as pltpu
from jax.experimental.pallas import tpu_sc as plsc

_N_POS_EMB = 100000

_NC = 2
_NS = 16
_NW = _NC * _NS
_LANES = 16

_CHUNK = 512
_GSUB = 128


def _make_gather(n, v, d):
    per_w = n // _NW
    n_chunks = per_w // _CHUNK
    assert n_chunks % 2 == 0
    v_main = (v // _NS) * _NS
    mesh = plsc.VectorSubcoreMesh(core_axis_name="c", subcore_axis_name="s")

    @functools.partial(
        pl.kernel,
        mesh=mesh,
        compiler_params=pltpu.CompilerParams(use_tc_tiling_on_sc=False),
        out_type=jax.ShapeDtypeStruct((n, d), jnp.float32),
        scratch_types=[
            pltpu.VMEM_SHARED((v, d), jnp.float32),
            [pltpu.VMEM((_CHUNK,), jnp.float32) for _ in range(2)],
            [pltpu.VMEM((_CHUNK,), jnp.int32) for _ in range(2)],
            [pltpu.VMEM((_CHUNK, d), jnp.float32) for _ in range(2)],
            [pltpu.SemaphoreType.DMA for _ in range(2)],
            [pltpu.SemaphoreType.DMA for _ in range(2)],
            [pltpu.SemaphoreType.DMA for _ in range(2)],
        ],
    )
    def gather_kernel(coord_hbm, table_hbm, out_hbm, table_s, coord_v, idx_v,
                      rows_v, csem, gsem, osem):
        sid = lax.axis_index("s")
        wid = sid * _NC + lax.axis_index("c")
        w_base = wid * per_w

        fill = v_main // _NS
        pltpu.sync_copy(table_hbm.at[pl.ds(sid * fill, fill)],
                        table_s.at[pl.ds(sid * fill, fill)])

        @pl.when(sid == 0)
        def _():
            pltpu.sync_copy(table_hbm.at[pl.ds(v_main, v - v_main)],
                            table_s.at[pl.ds(v_main, v - v_main)])

        plsc.subcore_barrier()

        def coord_load(g, b):
            pltpu.async_copy(
                coord_hbm.at[pl.ds(w_base + g * _CHUNK, _CHUNK)], coord_v[b],
                csem[b])

        def coord_wait(b):
            pltpu.make_async_copy(
                coord_hbm.at[pl.ds(w_base, _CHUNK)], coord_v[b],
                csem[b]).wait()

        def quantize(b):
            @plsc.parallel_loop(0, _CHUNK, step=_LANES, unroll=8)
            def _(i):
                c = coord_v[b][pl.ds(i, _LANES)]
                pos = c * jnp.float32(_N_POS_EMB)
                pos = jnp.maximum(pos, jnp.float32(0.0))
                pos = jnp.minimum(pos, jnp.float32(_N_POS_EMB))
                idx_v[b][pl.ds(i, _LANES)] = pos.astype(jnp.int32)

        def fire_gathers(b):
            for j in range(_CHUNK // _GSUB):
                pltpu.async_copy(
                    table_s.at[idx_v[b].at[pl.ds(j * _GSUB, _GSUB)]],
                    rows_v[b].at[pl.ds(j * _GSUB, _GSUB)],
                    gsem[b])

        def gather_wait(b):
            pltpu.make_async_copy(
                out_hbm.at[pl.ds(w_base, _CHUNK)], rows_v[b], gsem[b]).wait()

        def fire_scatter(g, b):
            pltpu.async_copy(
                rows_v[b], out_hbm.at[pl.ds(w_base + g * _CHUNK, _CHUNK)],
                osem[b])

        def scatter_wait(b):
            pltpu.make_async_copy(
                rows_v[b], out_hbm.at[pl.ds(w_base, _CHUNK)], osem[b]).wait()

        coord_load(0, 0)
        coord_load(1, 1)

        def pair_body(p, carry):
            for b in range(2):
                g = 2 * p + b

                coord_wait(b)

                @pl.when(g >= 2)
                def _():
                    scatter_wait(b)

                quantize(b)

                @pl.when(g + 2 < n_chunks)
                def _():
                    coord_load(g + 2, b)

                fire_gathers(b)

                @pl.when(g >= 1)
                def _():
                    gather_wait(1 - b)
                    fire_scatter(g - 1, 1 - b)
            return carry

        lax.fori_loop(0, n_chunks // 2, pair_body, 0)

        gather_wait(1)
        fire_scatter(n_chunks - 1, 1)
        scatter_wait(0)
        scatter_wait(1)

    return gather_kernel


def kernel(coord, embeddings_table, special_token):
    b, h = coord.shape
    d = embeddings_table.shape[1]
    n = b * h
    flat = coord.reshape(n)
    out = _make_gather(n, embeddings_table.shape[0], d)(flat, embeddings_table)
    return out.reshape(b, h, d)
